# Initial kernel scaffold; baseline (speedup 1.0000x reference)
import jax
import jax.numpy as jnp
from jax.experimental import pallas as pl


def kernel(c1_m, c1_b, c1_rsel, c1_csel, c2_m, c2_b, c2_rsel, c2_csel, c3_m, c3_b, c3_rsel, c3_csel, c4_m, c4_b, c4_rsel, c4_csel, fc1_w, fc1_b, fc2_w, fc2_b, x):
    raise NotImplementedError("write your pallas kernel here")


# speedup vs baseline: 47.4629x; 47.4629x over previous
import jax
import jax.numpy as jnp
from jax.experimental import pallas as pl
from jax.experimental.pallas import tpu as pltpu

_T = 256
_OUT = 1000


def _conv_pool(B, m_ref, b_ref, cs_ref, hp, t):
    acc = jnp.dot(B, m_ref[...], preferred_element_type=jnp.float32)
    o = (acc + b_ref[...]).astype(jnp.bfloat16)
    r = jnp.maximum(o[: hp * t], o[hp * t:])
    cp = jnp.dot(r, cs_ref[...], preferred_element_type=jnp.float32)
    w2 = cp.shape[1] // 2
    return jnp.maximum(cp[:, :w2], cp[:, w2:]).astype(jnp.bfloat16)


def _windows(P, hin, k, pad, hp_out, t):
    wc = P.shape[1]
    zero = jnp.zeros((t, wc), jnp.bfloat16)

    def row(j):
        return P[j * t:(j + 1) * t] if 0 <= j < hin else zero

    def win(i):
        return jnp.concatenate([row(i - pad + d) for d in range(k)], axis=1)

    order = [2 * i for i in range(hp_out)] + [2 * i + 1 for i in range(hp_out)]
    return jnp.concatenate([win(i) for i in order], axis=0)


def _fwd_kernel(x_ref,
                m1, b1, cs1,
                m2, b2, cs2,
                m3, b3, cs3,
                m4, b4, cs4,
                fw1, fb1, fw2, fb2,
                out_ref):
    t = x_ref.shape[0]
    xb = x_ref[...].astype(jnp.bfloat16)
    z = jnp.zeros((t, 64), jnp.bfloat16)
    xp = jnp.concatenate([z, xb, z], axis=1)

    order1 = [2 * i for i in range(16)] + [2 * i + 1 for i in range(16)]
    B1 = jnp.concatenate([xp[:, 32 * i: 32 * i + 160] for i in order1],
                         axis=0)
    P1 = _conv_pool(B1, m1, b1, cs1, 16, t)

    B2 = _windows(P1, 16, 5, 0, 6, t)
    P2 = _conv_pool(B2, m2, b2, cs2, 6, t)

    B3 = _windows(P2, 6, 3, 1, 3, t)
    P3 = _conv_pool(B3, m3, b3, cs3, 3, t)

    B4 = _windows(P3, 3, 3, 1, 1, t)
    f = _conv_pool(B4, m4, b4, cs4, 1, t)

    h = jnp.dot(f, fw1[...], preferred_element_type=jnp.float32) + fb1[...]
    y = jnp.dot(h.astype(jnp.bfloat16), fw2[...],
                preferred_element_type=jnp.float32) + fb2[...]
    out_ref[...] = y[:, :_OUT]


def _const_specs(arrays):
    return [pl.BlockSpec(a.shape, lambda i, _nd=a.ndim: (0,) * _nd)
            for a in arrays]


def kernel(c1_m, c1_b, c1_rsel, c1_csel,
           c2_m, c2_b, c2_rsel, c2_csel,
           c3_m, c3_b, c3_rsel, c3_csel,
           c4_m, c4_b, c4_rsel, c4_csel,
           fc1_w, fc1_b, fc2_w, fc2_b,
           x):
    n = x.shape[0]
    x2 = x.reshape(n, 32 * 32)
    n_pad = ((n + _T - 1) // _T) * _T
    if n_pad != n:
        x2 = jnp.concatenate(
            [x2, jnp.zeros((n_pad - n, 32 * 32), x2.dtype)], axis=0)

    consts = [
        c1_m.reshape(5 * 32, 192), c1_b, c1_csel,
        c2_m.reshape(5 * 96, 192), c2_b, c2_csel,
        c3_m.reshape(3 * 96, 192), c3_b, c3_csel,
        c4_m.reshape(3 * 96, 192), c4_b, c4_csel,
        fc1_w, fc1_b, fc2_w, fc2_b,
    ]
    weight_bytes = sum(int(a.size) * a.dtype.itemsize for a in consts)

    out = pl.pallas_call(
        _fwd_kernel,
        out_shape=jax.ShapeDtypeStruct((n_pad, _OUT), jnp.float32),
        grid=(n_pad // _T,),
        in_specs=[pl.BlockSpec((_T, 32 * 32), lambda i: (i, 0))]
                 + _const_specs(consts),
        out_specs=pl.BlockSpec((_T, _OUT), lambda i: (i, 0)),
        compiler_params=pltpu.CompilerParams(
            dimension_semantics=("parallel",),
            vmem_limit_bytes=64 * 1024 * 1024),
        cost_estimate=pl.CostEstimate(
            flops=7_500_000 * n_pad,
            transcendentals=0,
            bytes_accessed=weight_bytes + n_pad * (32 * 32 * 4 + _OUT * 4)),
    )(x2, *consts)
    return out[:n]


# speedup vs baseline: 52.4644x; 1.1054x over previous
import jax
import jax.numpy as jnp
from jax.experimental import pallas as pl
from jax.experimental.pallas import tpu as pltpu

_T = 512
_OUT = 1000


def _conv_pool(B, m_ref, b_ref, cs_ref, hp, t):
    acc = jnp.dot(B, m_ref[...], preferred_element_type=jnp.float32)
    o = (acc + b_ref[...]).astype(jnp.bfloat16)
    r = jnp.maximum(o[: hp * t], o[hp * t:])
    cp = jnp.dot(r, cs_ref[...], preferred_element_type=jnp.float32)
    w2 = cp.shape[1] // 2
    return jnp.maximum(cp[:, :w2], cp[:, w2:]).astype(jnp.bfloat16)


def _windows(P, hin, k, pad, hp_out, t):
    wc = P.shape[1]
    zero = jnp.zeros((t, wc), jnp.bfloat16)

    def row(j):
        return P[j * t:(j + 1) * t] if 0 <= j < hin else zero

    def win(i):
        return jnp.concatenate([row(i - pad + d) for d in range(k)], axis=1)

    order = [2 * i for i in range(hp_out)] + [2 * i + 1 for i in range(hp_out)]
    return jnp.concatenate([win(i) for i in order], axis=0)


def _fwd_kernel(x_ref,
                m1, b1, cs1,
                m2, b2, cs2,
                m3, b3, cs3,
                m4, b4, cs4,
                fw1, fb1, fw2, fb2,
                out_ref):
    t = x_ref.shape[0]
    xb = x_ref[...].astype(jnp.bfloat16)
    z = jnp.zeros((t, 64), jnp.bfloat16)
    xp = jnp.concatenate([z, xb, z], axis=1)

    order1 = [2 * i for i in range(16)] + [2 * i + 1 for i in range(16)]
    B1 = jnp.concatenate([xp[:, 32 * i: 32 * i + 160] for i in order1],
                         axis=0)
    P1 = _conv_pool(B1, m1, b1, cs1, 16, t)

    B2 = _windows(P1, 16, 5, 0, 6, t)
    P2 = _conv_pool(B2, m2, b2, cs2, 6, t)

    B3 = _windows(P2, 6, 3, 1, 3, t)
    P3 = _conv_pool(B3, m3, b3, cs3, 3, t)

    B4 = _windows(P3, 3, 3, 1, 1, t)
    f = _conv_pool(B4, m4, b4, cs4, 1, t)

    h = jnp.dot(f, fw1[...], preferred_element_type=jnp.float32) + fb1[...]
    y = jnp.dot(h.astype(jnp.bfloat16), fw2[...],
                preferred_element_type=jnp.float32) + fb2[...]
    out_ref[...] = y[:, :_OUT]


def _const_specs(arrays):
    return [pl.BlockSpec(a.shape, lambda i, _nd=a.ndim: (0,) * _nd)
            for a in arrays]


def kernel(c1_m, c1_b, c1_rsel, c1_csel,
           c2_m, c2_b, c2_rsel, c2_csel,
           c3_m, c3_b, c3_rsel, c3_csel,
           c4_m, c4_b, c4_rsel, c4_csel,
           fc1_w, fc1_b, fc2_w, fc2_b,
           x):
    n = x.shape[0]
    x2 = x.reshape(n, 32 * 32)
    n_pad = ((n + _T - 1) // _T) * _T
    if n_pad != n:
        x2 = jnp.concatenate(
            [x2, jnp.zeros((n_pad - n, 32 * 32), x2.dtype)], axis=0)

    consts = [
        c1_m.reshape(5 * 32, 192), c1_b, c1_csel,
        c2_m.reshape(5 * 96, 192), c2_b, c2_csel,
        c3_m.reshape(3 * 96, 192), c3_b, c3_csel,
        c4_m.reshape(3 * 96, 192), c4_b, c4_csel,
        fc1_w, fc1_b, fc2_w, fc2_b,
    ]
    weight_bytes = sum(int(a.size) * a.dtype.itemsize for a in consts)

    out = pl.pallas_call(
        _fwd_kernel,
        out_shape=jax.ShapeDtypeStruct((n_pad, _OUT), jnp.float32),
        grid=(n_pad // _T,),
        in_specs=[pl.BlockSpec((_T, 32 * 32), lambda i: (i, 0))]
                 + _const_specs(consts),
        out_specs=pl.BlockSpec((_T, _OUT), lambda i: (i, 0)),
        compiler_params=pltpu.CompilerParams(
            dimension_semantics=("parallel",),
            vmem_limit_bytes=64 * 1024 * 1024),
        cost_estimate=pl.CostEstimate(
            flops=7_500_000 * n_pad,
            transcendentals=0,
            bytes_accessed=weight_bytes + n_pad * (32 * 32 * 4 + _OUT * 4)),
    )(x2, *consts)
    return out[:n]


# speedup vs baseline: 53.8398x; 1.0262x over previous
import jax
import jax.numpy as jnp
from jax.experimental import pallas as pl
from jax.experimental.pallas import tpu as pltpu

_T = 1024
_OUT = 1000


def _conv_pool(B, m_ref, b_ref, cs_ref, hp, t):
    acc = jnp.dot(B, m_ref[...], preferred_element_type=jnp.float32)
    o = (acc + b_ref[...]).astype(jnp.bfloat16)
    r = jnp.maximum(o[: hp * t], o[hp * t:])
    cp = jnp.dot(r, cs_ref[...], preferred_element_type=jnp.float32)
    w2 = cp.shape[1] // 2
    return jnp.maximum(cp[:, :w2], cp[:, w2:]).astype(jnp.bfloat16)


def _windows(P, hin, k, pad, hp_out, t):
    wc = P.shape[1]
    zero = jnp.zeros((t, wc), jnp.bfloat16)

    def row(j):
        return P[j * t:(j + 1) * t] if 0 <= j < hin else zero

    def win(i):
        return jnp.concatenate([row(i - pad + d) for d in range(k)], axis=1)

    order = [2 * i for i in range(hp_out)] + [2 * i + 1 for i in range(hp_out)]
    return jnp.concatenate([win(i) for i in order], axis=0)


def _fwd_kernel(x_ref,
                m1, b1, cs1,
                m2, b2, cs2,
                m3, b3, cs3,
                m4, b4, cs4,
                fw1, fb1, fw2, fb2,
                out_ref):
    t = x_ref.shape[0]
    xb = x_ref[...].astype(jnp.bfloat16)
    z = jnp.zeros((t, 64), jnp.bfloat16)
    xp = jnp.concatenate([z, xb, z], axis=1)

    order1 = [2 * i for i in range(16)] + [2 * i + 1 for i in range(16)]
    B1 = jnp.concatenate([xp[:, 32 * i: 32 * i + 160] for i in order1],
                         axis=0)
    P1 = _conv_pool(B1, m1, b1, cs1, 16, t)

    B2 = _windows(P1, 16, 5, 0, 6, t)
    P2 = _conv_pool(B2, m2, b2, cs2, 6, t)

    B3 = _windows(P2, 6, 3, 1, 3, t)
    P3 = _conv_pool(B3, m3, b3, cs3, 3, t)

    B4 = _windows(P3, 3, 3, 1, 1, t)
    f = _conv_pool(B4, m4, b4, cs4, 1, t)

    h = jnp.dot(f, fw1[...], preferred_element_type=jnp.float32) + fb1[...]
    y = jnp.dot(h.astype(jnp.bfloat16), fw2[...],
                preferred_element_type=jnp.float32) + fb2[...]
    out_ref[...] = y[:, :_OUT]


def _const_specs(arrays):
    return [pl.BlockSpec(a.shape, lambda i, _nd=a.ndim: (0,) * _nd)
            for a in arrays]


def kernel(c1_m, c1_b, c1_rsel, c1_csel,
           c2_m, c2_b, c2_rsel, c2_csel,
           c3_m, c3_b, c3_rsel, c3_csel,
           c4_m, c4_b, c4_rsel, c4_csel,
           fc1_w, fc1_b, fc2_w, fc2_b,
           x):
    n = x.shape[0]
    x2 = x.reshape(n, 32 * 32)
    n_pad = ((n + _T - 1) // _T) * _T
    if n_pad != n:
        x2 = jnp.concatenate(
            [x2, jnp.zeros((n_pad - n, 32 * 32), x2.dtype)], axis=0)

    consts = [
        c1_m.reshape(5 * 32, 192), c1_b, c1_csel,
        c2_m.reshape(5 * 96, 192), c2_b, c2_csel,
        c3_m.reshape(3 * 96, 192), c3_b, c3_csel,
        c4_m.reshape(3 * 96, 192), c4_b, c4_csel,
        fc1_w, fc1_b, fc2_w, fc2_b,
    ]
    weight_bytes = sum(int(a.size) * a.dtype.itemsize for a in consts)

    out = pl.pallas_call(
        _fwd_kernel,
        out_shape=jax.ShapeDtypeStruct((n_pad, _OUT), jnp.float32),
        grid=(n_pad // _T,),
        in_specs=[pl.BlockSpec((_T, 32 * 32), lambda i: (i, 0))]
                 + _const_specs(consts),
        out_specs=pl.BlockSpec((_T, _OUT), lambda i: (i, 0)),
        compiler_params=pltpu.CompilerParams(
            dimension_semantics=("parallel",),
            vmem_limit_bytes=64 * 1024 * 1024),
        cost_estimate=pl.CostEstimate(
            flops=7_500_000 * n_pad,
            transcendentals=0,
            bytes_accessed=weight_bytes + n_pad * (32 * 32 * 4 + _OUT * 4)),
    )(x2, *consts)
    return out[:n]


# speedup vs baseline: 55.5462x; 1.0317x over previous
import jax
import jax.numpy as jnp
from jax.experimental import pallas as pl
from jax.experimental.pallas import tpu as pltpu

_T = 1024
_OUT = 1000
_N = 192


def _pair_weight(m, stride):
    k, wc, n = m.shape
    core = jnp.pad(m, ((0, 0), (0, stride - wc), (0, 0))).reshape(k * stride, n)
    even = jnp.pad(core, ((0, stride), (0, 0)))
    odd = jnp.pad(core, ((stride, 0), (0, 0)))
    return jnp.concatenate([even, odd], axis=1)


def _pad_csel(cs):
    h = cs.shape[1] // 2
    z = jnp.zeros((cs.shape[0], 128 - h), cs.dtype)
    return jnp.concatenate([cs[:, :h], z, cs[:, h:], z], axis=1)


def _conv_pair_pool(B, w_ref, b_ref, cs_ref):
    acc = jnp.dot(B, w_ref[...], preferred_element_type=jnp.float32)
    o = (acc + b_ref[...]).astype(jnp.bfloat16)
    r = jnp.maximum(o[:, :_N], o[:, _N:])
    cp = jnp.dot(r, cs_ref[...], preferred_element_type=jnp.float32)
    return jnp.maximum(cp[:, :128], cp[:, 128:]).astype(jnp.bfloat16)


def _pair_windows(P, hin, k, pad, hp, t):
    zero = jnp.zeros((t, 128), jnp.bfloat16)

    def row(j):
        return P[j * t:(j + 1) * t] if 0 <= j < hin else zero

    wins = [jnp.concatenate([row(2 * i - pad + d) for d in range(k + 1)],
                            axis=1) for i in range(hp)]
    return jnp.concatenate(wins, axis=0)


def _fwd_kernel(x_ref,
                w1, b1, cs1,
                w2, b2, cs2,
                w3, b3, cs3,
                w4, b4, cs4,
                fw1, fb1, fw2, fb2,
                out_ref):
    t = x_ref.shape[0]
    xb = x_ref[...].astype(jnp.bfloat16)
    z = jnp.zeros((t, 64), jnp.bfloat16)
    xp = jnp.concatenate([z, xb, z], axis=1)

    B1 = jnp.concatenate([xp[:, 64 * i: 64 * i + 192] for i in range(16)],
                         axis=0)
    P1 = _conv_pair_pool(B1, w1, b1, cs1)

    B2 = _pair_windows(P1, 16, 5, 0, 6, t)
    P2 = _conv_pair_pool(B2, w2, b2, cs2)

    B3 = _pair_windows(P2, 6, 3, 1, 3, t)
    P3 = _conv_pair_pool(B3, w3, b3, cs3)

    B4 = _pair_windows(P3, 3, 3, 1, 1, t)
    f = _conv_pair_pool(B4, w4, b4, cs4)

    h = jnp.dot(f, fw1[...], preferred_element_type=jnp.float32) + fb1[...]
    y = jnp.dot(h.astype(jnp.bfloat16), fw2[...],
                preferred_element_type=jnp.float32) + fb2[...]
    out_ref[...] = y[:, :_OUT]


def _const_specs(arrays):
    return [pl.BlockSpec(a.shape, lambda i, _nd=a.ndim: (0,) * _nd)
            for a in arrays]


def kernel(c1_m, c1_b, c1_rsel, c1_csel,
           c2_m, c2_b, c2_rsel, c2_csel,
           c3_m, c3_b, c3_rsel, c3_csel,
           c4_m, c4_b, c4_rsel, c4_csel,
           fc1_w, fc1_b, fc2_w, fc2_b,
           x):
    n = x.shape[0]
    x2 = x.reshape(n, 32 * 32)
    n_pad = ((n + _T - 1) // _T) * _T
    if n_pad != n:
        x2 = jnp.concatenate(
            [x2, jnp.zeros((n_pad - n, 32 * 32), x2.dtype)], axis=0)

    def bias2(b):
        return jnp.concatenate([b, b], axis=1)

    consts = [
        _pair_weight(c1_m, 32), bias2(c1_b), _pad_csel(c1_csel),
        _pair_weight(c2_m, 128), bias2(c2_b), _pad_csel(c2_csel),
        _pair_weight(c3_m, 128), bias2(c3_b), _pad_csel(c3_csel),
        _pair_weight(c4_m, 128), bias2(c4_b), _pad_csel(c4_csel),
        jnp.pad(fc1_w, ((0, 64), (0, 0))), fc1_b,
        fc2_w, fc2_b,
    ]
    weight_bytes = sum(int(a.size) * a.dtype.itemsize for a in consts)

    out = pl.pallas_call(
        _fwd_kernel,
        out_shape=jax.ShapeDtypeStruct((n_pad, _OUT), jnp.float32),
        grid=(n_pad // _T,),
        in_specs=[pl.BlockSpec((_T, 32 * 32), lambda i: (i, 0))]
                 + _const_specs(consts),
        out_specs=pl.BlockSpec((_T, _OUT), lambda i: (i, 0)),
        compiler_params=pltpu.CompilerParams(
            dimension_semantics=("parallel",),
            vmem_limit_bytes=64 * 1024 * 1024),
        cost_estimate=pl.CostEstimate(
            flops=7_500_000 * n_pad,
            transcendentals=0,
            bytes_accessed=weight_bytes + n_pad * (32 * 32 * 4 + _OUT * 4)),
    )(x2, *consts)
    return out[:n]
